# baseline (speedup 1.0000x reference)
import jax
import jax.numpy as jnp
from jax.experimental import pallas as pl
from jax.experimental.pallas import tpu as pltpu


_BLOCK_ROWS = 8192


def _copy_kernel(src_ref, dst_ref):
    dst_ref[...] = src_ref[...]


def kernel(prototypes):
    rows = prototypes.shape[0]
    return pl.pallas_call(
        _copy_kernel,
        out_shape=jax.ShapeDtypeStruct(prototypes.shape, prototypes.dtype),
        grid=(rows // _BLOCK_ROWS,),
        in_specs=[pl.BlockSpec((_BLOCK_ROWS, prototypes.shape[1]), lambda i: (i, 0))],
        out_specs=pl.BlockSpec((_BLOCK_ROWS, prototypes.shape[1]), lambda i: (i, 0)),
    )(prototypes)


# speedup vs baseline: 1.0726x; 1.0726x over previous
import jax
import jax.numpy as jnp
from jax.experimental import pallas as pl
from jax.experimental.pallas import tpu as pltpu


_NUM_CHUNKS = 4


def _copy_kernel(src_ref, dst_ref, buf, in_sems, out_sems):
    rows = src_ref.shape[0]
    chunk = rows // _NUM_CHUNKS
    ins, outs = [], []
    for i in range(_NUM_CHUNKS):
        c = pltpu.make_async_copy(
            src_ref.at[pl.ds(i * chunk, chunk)], buf.at[i], in_sems.at[i]
        )
        c.start()
        ins.append(c)
    for i in range(_NUM_CHUNKS):
        ins[i].wait()
        c = pltpu.make_async_copy(
            buf.at[i], dst_ref.at[pl.ds(i * chunk, chunk)], out_sems.at[i]
        )
        c.start()
        outs.append(c)
    for c in outs:
        c.wait()


def kernel(prototypes):
    rows, feat = prototypes.shape
    chunk = rows // _NUM_CHUNKS
    return pl.pallas_call(
        _copy_kernel,
        out_shape=jax.ShapeDtypeStruct(prototypes.shape, prototypes.dtype),
        in_specs=[pl.BlockSpec(memory_space=pl.ANY)],
        out_specs=pl.BlockSpec(memory_space=pl.ANY),
        scratch_shapes=[
            pltpu.VMEM((_NUM_CHUNKS, chunk, feat), prototypes.dtype),
            pltpu.SemaphoreType.DMA((_NUM_CHUNKS,)),
            pltpu.SemaphoreType.DMA((_NUM_CHUNKS,)),
        ],
    )(prototypes)


# speedup vs baseline: 1.1726x; 1.0932x over previous
import jax
import jax.numpy as jnp
from jax.experimental import pallas as pl
from jax.experimental.pallas import tpu as pltpu


_NUM_CHUNKS = 2


def _copy_kernel(src_ref, dst_ref, buf, in_sems, out_sems):
    rows = src_ref.shape[0]
    chunk = rows // _NUM_CHUNKS
    ins, outs = [], []
    for i in range(_NUM_CHUNKS):
        c = pltpu.make_async_copy(
            src_ref.at[pl.ds(i * chunk, chunk)], buf.at[i], in_sems.at[i]
        )
        c.start()
        ins.append(c)
    for i in range(_NUM_CHUNKS):
        ins[i].wait()
        c = pltpu.make_async_copy(
            buf.at[i], dst_ref.at[pl.ds(i * chunk, chunk)], out_sems.at[i]
        )
        c.start()
        outs.append(c)
    for c in outs:
        c.wait()


def kernel(prototypes):
    rows, feat = prototypes.shape
    chunk = rows // _NUM_CHUNKS
    return pl.pallas_call(
        _copy_kernel,
        out_shape=jax.ShapeDtypeStruct(prototypes.shape, prototypes.dtype),
        in_specs=[pl.BlockSpec(memory_space=pl.ANY)],
        out_specs=pl.BlockSpec(memory_space=pl.ANY),
        scratch_shapes=[
            pltpu.VMEM((_NUM_CHUNKS, chunk, feat), prototypes.dtype),
            pltpu.SemaphoreType.DMA((_NUM_CHUNKS,)),
            pltpu.SemaphoreType.DMA((_NUM_CHUNKS,)),
        ],
    )(prototypes)
